# Initial kernel scaffold; baseline (speedup 1.0000x reference)
import functools

import jax
import jax.numpy as jnp
from jax import lax
from jax.experimental import pallas as pl
from jax.experimental.pallas import tpu as pltpu
from jax.experimental.pallas import tpu_sc as plsc

VOCAB = 100000
DEPTH = 64
B_TOTAL = 4096 * 200
CHUNK = 128
N_CHUNKS = B_TOTAL // CHUNK
NC, NS = 2, 16
NW = NC * NS
CPW = N_CHUNKS // NW


def _make_gather():
    mesh = plsc.VectorSubcoreMesh(core_axis_name="c", subcore_axis_name="s")

    @functools.partial(
        pl.kernel,
        mesh=mesh,
        out_type=jax.ShapeDtypeStruct((N_CHUNKS, CHUNK, DEPTH), jnp.float32),
        scratch_types=[
            pltpu.VMEM((CPW, CHUNK), jnp.int32),
            pltpu.VMEM((CHUNK, DEPTH), jnp.float32),
            pltpu.SemaphoreType.DMA,
        ],
    )
    def gather_kernel(table_hbm, idx_hbm, out_hbm, idx_v, rows_v, sem):
        wid = lax.axis_index("s") * NC + lax.axis_index("c")
        base = wid * CPW
        pltpu.sync_copy(idx_hbm.at[pl.ds(base, CPW)], idx_v)

        def step(j, carry):
            pltpu.async_copy(table_hbm.at[idx_v.at[j]], rows_v, sem).wait()
            pltpu.sync_copy(rows_v, out_hbm.at[base + j])
            return carry

        lax.fori_loop(0, CPW, step, 0)

    return gather_kernel


_gather = _make_gather()


@jax.jit
def kernel(x, embeddings):
    idx = x.reshape(N_CHUNKS, CHUNK)
    rows = _gather(embeddings, idx)
    return rows.reshape(x.shape[0], x.shape[1], DEPTH)


# speedup vs baseline: 3.5421x; 3.5421x over previous
import functools

import jax
import jax.numpy as jnp
from jax import lax
from jax.experimental import pallas as pl
from jax.experimental.pallas import tpu as pltpu
from jax.experimental.pallas import tpu_sc as plsc

VOCAB = 100000
DEPTH = 64
B_TOTAL = 4096 * 200
CHUNK = 128
N_CHUNKS = B_TOTAL // CHUNK
NC, NS = 2, 16
NW = NC * NS
CPW = N_CHUNKS // NW


def _make_gather():
    mesh = plsc.VectorSubcoreMesh(core_axis_name="c", subcore_axis_name="s")

    @functools.partial(
        pl.kernel,
        mesh=mesh,
        out_type=jax.ShapeDtypeStruct((N_CHUNKS, CHUNK, DEPTH), jnp.float32),
        scratch_types=[
            pltpu.VMEM((CPW, CHUNK), jnp.int32),
            pltpu.VMEM((CHUNK, DEPTH), jnp.float32),
            pltpu.SemaphoreType.DMA,
        ],
        compiler_params=pltpu.CompilerParams(use_tc_tiling_on_sc=False),
    )
    def gather_kernel(table_hbm, idx_hbm, out_hbm, idx_v, rows_v, sem):
        wid = lax.axis_index("s") * NC + lax.axis_index("c")
        base = wid * CPW
        pltpu.sync_copy(idx_hbm.at[pl.ds(base, CPW)], idx_v)

        def step(j, carry):
            pltpu.async_copy(table_hbm.at[idx_v.at[j]], rows_v, sem).wait()
            pltpu.sync_copy(rows_v, out_hbm.at[base + j])
            return carry

        lax.fori_loop(0, CPW, step, 0)

    return gather_kernel


_gather = _make_gather()


@jax.jit
def kernel(x, embeddings):
    idx = x.reshape(N_CHUNKS, CHUNK)
    rows = _gather(embeddings, idx)
    return rows.reshape(x.shape[0], x.shape[1], DEPTH)


# speedup vs baseline: 4.2349x; 1.1956x over previous
import functools

import jax
import jax.numpy as jnp
from jax import lax
from jax.experimental import pallas as pl
from jax.experimental.pallas import tpu as pltpu
from jax.experimental.pallas import tpu_sc as plsc

VOCAB = 100000
DEPTH = 64
B_TOTAL = 4096 * 200
CHUNK = 128
N_CHUNKS = B_TOTAL // CHUNK
NC, NS = 2, 16
NW = NC * NS
CPW = N_CHUNKS // NW


NBUF = 8
GROUPS = CPW // NBUF


def _make_gather():
    mesh = plsc.VectorSubcoreMesh(core_axis_name="c", subcore_axis_name="s")

    @functools.partial(
        pl.kernel,
        mesh=mesh,
        out_type=jax.ShapeDtypeStruct((N_CHUNKS, CHUNK, DEPTH), jnp.float32),
        scratch_types=[
            pltpu.VMEM((CPW, CHUNK), jnp.int32),
            [pltpu.VMEM((CHUNK, DEPTH), jnp.float32) for _ in range(NBUF)],
            [pltpu.SemaphoreType.DMA for _ in range(NBUF)],
            [pltpu.SemaphoreType.DMA for _ in range(NBUF)],
        ],
        compiler_params=pltpu.CompilerParams(use_tc_tiling_on_sc=False),
    )
    def gather_kernel(table_hbm, idx_hbm, out_hbm, idx_v, rows, gsem, ssem):
        wid = lax.axis_index("s") * NC + lax.axis_index("c")
        base = wid * CPW
        pltpu.sync_copy(idx_hbm.at[pl.ds(base, CPW)], idx_v)

        def gather(j, b):
            pltpu.make_async_copy(
                table_hbm.at[idx_v.at[j]], rows[b], gsem[b]
            ).start()

        def store(j, b):
            pltpu.make_async_copy(rows[b], out_hbm.at[base + j], ssem[b]).start()

        for b in range(NBUF):
            gather(b, b)

        def group(g, carry):
            for b in range(NBUF):
                j = g * NBUF + b
                pltpu.make_async_copy(
                    table_hbm.at[idx_v.at[0]], rows[b], gsem[b]
                ).wait()
                store(j, b)
            for b in range(NBUF):
                pltpu.make_async_copy(
                    rows[b], out_hbm.at[base], ssem[b]
                ).wait()
                gather((g + 1) * NBUF + b, b)
            return carry

        lax.fori_loop(0, GROUPS - 1, group, 0)

        for b in range(NBUF):
            j = (GROUPS - 1) * NBUF + b
            pltpu.make_async_copy(
                table_hbm.at[idx_v.at[0]], rows[b], gsem[b]
            ).wait()
            store(j, b)
        for b in range(NBUF):
            pltpu.make_async_copy(rows[b], out_hbm.at[base], ssem[b]).wait()

    return gather_kernel


_gather = _make_gather()


@jax.jit
def kernel(x, embeddings):
    idx = x.reshape(N_CHUNKS, CHUNK)
    rows = _gather(embeddings, idx)
    return rows.reshape(x.shape[0], x.shape[1], DEPTH)
